# baseline (speedup 1.0000x reference)
import dataclasses
import functools

import jax
import jax.numpy as jnp
from jax import lax
from jax.experimental import pallas as pl
from jax.experimental.pallas import tpu as pltpu
from jax.experimental.pallas import tpu_sc as plsc

_NSC = 2
_NSUB = 16
_L = 16
_G = 128
_CHUNK = 128


def _make_sc_propagate(N_pad, NSUP, SB, Dh):
    nodes_per = N_pad // _NSUB
    mesh = plsc.VectorSubcoreMesh(
        core_axis_name="c", subcore_axis_name="s",
        num_cores=_NSC, num_subcores=_NSUB)

    def body(xs_hbm, row_hbm, col_hbm, w_hbm, stage_hbm, dis_hbm,
             src_sh, acc_sh, deg_sh,
             row_s, col_s, w_s, sl_t, gbuf):
        c = lax.axis_index("c")
        s = lax.axis_index("s")
        nb = s * nodes_per
        sb_off = c * N_pad + nb

        def zero_gbuf():
            @pl.loop(0, _CHUNK)
            def _(i):
                for k in range(Dh // _L):
                    gbuf[i, pl.ds(k * _L, _L)] = jnp.zeros((_L,), jnp.float32)

        def zero_spmem_slice(dst):
            for k in range(nodes_per // _CHUNK):
                pltpu.sync_copy(gbuf, dst.at[pl.ds(nb + k * _CHUNK, _CHUNK)])

        def scale_to_stage(src_hbm_or_spmem, from_spmem, squared):
            for k in range(nodes_per // _CHUNK):
                if from_spmem:
                    pltpu.sync_copy(
                        src_hbm_or_spmem.at[pl.ds(nb + k * _CHUNK, _CHUNK)],
                        gbuf)
                else:
                    pltpu.sync_copy(
                        src_hbm_or_spmem.at[c, pl.ds(nb + k * _CHUNK, _CHUNK)],
                        gbuf)

                @pl.loop(0, _CHUNK)
                def _(i):
                    dv = plsc.load_gather(
                        sl_t, [jnp.full((_L,), k * _CHUNK, jnp.int32) + i])
                    dv = dv * dv if squared else dv
                    for q in range(Dh // _L):
                        g = gbuf[i, pl.ds(q * _L, _L)]
                        gbuf[i, pl.ds(q * _L, _L)] = g * dv

                pltpu.sync_copy(
                    gbuf, stage_hbm.at[pl.ds(sb_off + k * _CHUNK, _CHUNK)])

        zero_gbuf()
        zero_spmem_slice(acc_sh)
        zero_spmem_slice(src_sh)

        @pl.loop(0, nodes_per, step=_L)
        def _(i):
            sl_t[pl.ds(i, _L)] = jnp.zeros((_L,), jnp.float32)

        pltpu.sync_copy(sl_t, deg_sh.at[pl.ds(nb, nodes_per)])
        plsc.subcore_barrier()

        @pl.loop(0, NSUP)
        def _(sc_i):
            pltpu.sync_copy(col_hbm.at[s, sc_i], col_s)
            pltpu.sync_copy(w_hbm.at[s, sc_i], w_s)
            for b in range(SB):
                pltpu.sync_copy(w_s.at[pl.ds(b * _CHUNK, _CHUNK)],
                                deg_sh.at[col_s.at[b]], add=True)

        plsc.subcore_barrier()

        pltpu.sync_copy(deg_sh.at[pl.ds(nb, nodes_per)], sl_t)

        @pl.loop(0, nodes_per, step=_L)
        def _(i):
            v = sl_t[pl.ds(i, _L)]
            bi = plsc.bitcast(v, jnp.int32)
            bi = jnp.full((_L,), 0x5F3759DF, jnp.int32) - lax.shift_right_logical(
                bi, jnp.full((_L,), 1, jnp.int32))
            y = plsc.bitcast(bi, jnp.float32)
            for _ in range(4):
                y = y * (1.5 - 0.5 * v * y * y)
            sl_t[pl.ds(i, _L)] = y

        pltpu.sync_copy(sl_t, dis_hbm.at[c, pl.ds(nb, nodes_per)])
        scale_to_stage(xs_hbm, from_spmem=False, squared=False)
        plsc.subcore_barrier()

        def hop(dst):
            @pl.loop(0, NSUP)
            def _(sc_i):
                pltpu.sync_copy(row_hbm.at[s, sc_i], row_s)
                pltpu.sync_copy(col_hbm.at[s, sc_i], col_s)
                pltpu.sync_copy(w_hbm.at[s, sc_i], w_s)
                @pl.loop(0, SB * _CHUNK, step=_L)
                def _(k):
                    b2, k2 = k // _CHUNK, k % _CHUNK
                    r = row_s[b2, pl.ds(k2, _L)]
                    row_s[b2, pl.ds(k2, _L)] = r + c * N_pad

                for b in range(SB):
                    pltpu.sync_copy(stage_hbm.at[row_s.at[b]], gbuf)

                    @pl.loop(0, _CHUNK)
                    def _(e):
                        nv = plsc.load_gather(
                            w_s, [jnp.full((_L,), b * _CHUNK, jnp.int32) + e])
                        for q in range(Dh // _L):
                            g = gbuf[e, pl.ds(q * _L, _L)]
                            gbuf[e, pl.ds(q * _L, _L)] = g * nv

                    pltpu.sync_copy(gbuf, dst.at[col_s.at[b]], add=True)

        hop(acc_sh)
        plsc.subcore_barrier()
        scale_to_stage(acc_sh, from_spmem=True, squared=True)
        plsc.subcore_barrier()
        hop(src_sh)
        plsc.subcore_barrier()
        pltpu.sync_copy(src_sh.at[pl.ds(nb, nodes_per)],
                        stage_hbm.at[pl.ds(sb_off, nodes_per)])

    cp = pltpu.CompilerParams()
    if "needs_layout_passes" in pltpu.CompilerParams.__dataclass_fields__:
        cp = dataclasses.replace(cp, needs_layout_passes=False)
    if "use_tc_tiling_on_sc" in pltpu.CompilerParams.__dataclass_fields__:
        cp = dataclasses.replace(cp, use_tc_tiling_on_sc=False)
    return pl.kernel(
        body,
        out_type=(jax.ShapeDtypeStruct((_NSC * N_pad, Dh), jnp.float32),
                  jax.ShapeDtypeStruct((_NSC, N_pad), jnp.float32)),
        mesh=mesh,
        compiler_params=cp,
        scratch_types=[
            pltpu.VMEM_SHARED((N_pad, Dh), jnp.float32),
            pltpu.VMEM_SHARED((N_pad, Dh), jnp.float32),
            pltpu.VMEM_SHARED((N_pad,), jnp.float32),
            pltpu.VMEM((SB, _CHUNK), jnp.int32),
            pltpu.VMEM((SB, _CHUNK), jnp.int32),
            pltpu.VMEM((SB * _CHUNK,), jnp.float32),
            pltpu.VMEM((N_pad // _NSUB,), jnp.float32),
            pltpu.VMEM((_CHUNK, Dh), jnp.float32),
        ],
    )


def _tc_body(NB, hs_ref, b_ref, dis_ref, W1a_ref, W1b_ref, b1_ref, W2_ref,
             b2_ref, out_ref, acc_s, acc_c):
    i = pl.program_id(0)

    @pl.when(i == 0)
    def _():
        acc_s[...] = jnp.zeros_like(acc_s)
        acc_c[...] = jnp.zeros_like(acc_c)

    dv = dis_ref[0, 0, :][:, None]
    y = (jnp.dot(hs_ref[0] * dv, W1a_ref[...],
                 preferred_element_type=jnp.float32)
         + jnp.dot(hs_ref[1] * dv, W1b_ref[...],
                   preferred_element_type=jnp.float32))
    bt = b_ref[0, 0, :]
    onehot_t = (lax.broadcasted_iota(jnp.int32, (_G, bt.shape[0]), 0)
                == bt[None, :]).astype(jnp.float32)
    acc_s[...] += jnp.dot(onehot_t, y, preferred_element_type=jnp.float32)
    acc_c[...] += jnp.sum(onehot_t, axis=1, keepdims=True)

    @pl.when(i == NB - 1)
    def _():
        pooled = acc_s[...] / jnp.maximum(acc_c[...], 1.0) + b1_ref[...]
        out_ref[...] = (jnp.dot(pooled, W2_ref[...],
                                preferred_element_type=jnp.float32)
                        + b2_ref[...])


def kernel(x, edge_index, edge_attr, batch, W1, b1, W2, b2):
    N, D = x.shape
    E = edge_index.shape[1]
    H = W1.shape[1]
    P = W2.shape[1]
    Dh = D // 2

    N_pad = -(-N // 256) * 256
    SB = 8
    per = _NSUB * SB * _CHUNK
    E2 = E + N
    E2p = -(-E2 // per) * per
    NSUP = E2p // per

    loop = jnp.arange(N, dtype=jnp.int32)
    row = jnp.concatenate([edge_index[0], loop])
    col = jnp.concatenate([edge_index[1], loop])
    w = jnp.concatenate([edge_attr, jnp.ones((N,), x.dtype)])
    padn = E2p - E2
    shape4 = (_NSUB, NSUP, SB, _CHUNK)
    row4 = jnp.pad(row, (0, padn)).reshape(shape4)
    col4 = jnp.pad(col, (0, padn)).reshape(shape4)
    w3 = jnp.pad(w, (0, padn)).reshape(_NSUB, NSUP, SB * _CHUNK)

    x_pad = jnp.pad(x, ((0, N_pad - N), (0, 0)))
    xs = x_pad.reshape(N_pad, _NSC, Dh).transpose(1, 0, 2)

    stage, dis = _make_sc_propagate(N_pad, NSUP, SB, Dh)(xs, row4, col4, w3)
    h2 = stage.reshape(_NSC, N_pad, Dh)

    BN = 1024
    NB = N_pad // BN
    batch3 = jnp.pad(batch, (0, N_pad - N), constant_values=_G).reshape(
        NB, 1, BN)
    dis3 = dis[0].reshape(NB, 1, BN)
    W1a, W1b = W1[:Dh], W1[Dh:]

    out = pl.pallas_call(
        functools.partial(_tc_body, NB),
        grid=(NB,),
        in_specs=[
            pl.BlockSpec((_NSC, BN, Dh), lambda i: (0, i, 0)),
            pl.BlockSpec((1, 1, BN), lambda i: (i, 0, 0)),
            pl.BlockSpec((1, 1, BN), lambda i: (i, 0, 0)),
            pl.BlockSpec((Dh, H), lambda i: (0, 0)),
            pl.BlockSpec((Dh, H), lambda i: (0, 0)),
            pl.BlockSpec((1, H), lambda i: (0, 0)),
            pl.BlockSpec((H, P), lambda i: (0, 0)),
            pl.BlockSpec((1, P), lambda i: (0, 0)),
        ],
        out_specs=pl.BlockSpec((_G, P), lambda i: (0, 0)),
        out_shape=jax.ShapeDtypeStruct((_G, P), jnp.float32),
        scratch_shapes=[
            pltpu.VMEM((_G, H), jnp.float32),
            pltpu.VMEM((_G, 1), jnp.float32),
        ],
    )(h2, batch3, dis3, W1a, W1b, b1.reshape(1, H), W2, b2.reshape(1, P))
    return out


# speedup vs baseline: 2.0877x; 2.0877x over previous
import dataclasses
import functools

import jax
import jax.numpy as jnp
from jax import lax
from jax.experimental import pallas as pl
from jax.experimental.pallas import tpu as pltpu
from jax.experimental.pallas import tpu_sc as plsc

_NSC = 2
_NSUB = 16
_L = 16
_G = 128
_CHUNK = 128


def _make_sc_propagate(N_pad, NSUP, SB, Dh):
    nodes_per = N_pad // _NSUB
    mesh = plsc.VectorSubcoreMesh(
        core_axis_name="c", subcore_axis_name="s",
        num_cores=_NSC, num_subcores=_NSUB)

    def body(xs_hbm, row_hbm, col_hbm, w_hbm, stage_hbm, dis_hbm,
             src_sh, acc_sh, deg_sh,
             row_s, col_s, w_s, sl_t, gbuf, ssems):
        c = lax.axis_index("c")
        s = lax.axis_index("s")
        nb = s * nodes_per
        sb_off = c * N_pad + nb
        EPS = SB * _CHUNK

        def zero_gbuf_rows(n):
            @pl.loop(0, n)
            def _(i):
                for k in range(Dh // _L):
                    gbuf[i, pl.ds(k * _L, _L)] = jnp.zeros((_L,), jnp.float32)

        def zero_spmem_slice(dst):
            for k in range(nodes_per // _CHUNK):
                pltpu.sync_copy(gbuf.at[pl.ds(0, _CHUNK)],
                                dst.at[pl.ds(nb + k * _CHUNK, _CHUNK)])

        def scale_slice(dst, squared):
            for k in range(nodes_per // _CHUNK):
                pltpu.sync_copy(dst.at[pl.ds(nb + k * _CHUNK, _CHUNK)],
                                gbuf.at[pl.ds(0, _CHUNK)])

                @pl.loop(0, _CHUNK)
                def _(i):
                    dv = plsc.load_gather(
                        sl_t, [jnp.full((_L,), k * _CHUNK, jnp.int32) + i])
                    dv = dv * dv if squared else dv
                    for q in range(Dh // _L):
                        g = gbuf[i, pl.ds(q * _L, _L)]
                        gbuf[i, pl.ds(q * _L, _L)] = g * dv

                pltpu.sync_copy(gbuf.at[pl.ds(0, _CHUNK)],
                                dst.at[pl.ds(nb + k * _CHUNK, _CHUNK)])

        pltpu.sync_copy(xs_hbm.at[c, pl.ds(nb, nodes_per)],
                        src_sh.at[pl.ds(nb, nodes_per)])

        zero_gbuf_rows(_CHUNK)
        zero_spmem_slice(acc_sh)

        @pl.loop(0, nodes_per, step=_L)
        def _(i):
            sl_t[pl.ds(i, _L)] = jnp.zeros((_L,), jnp.float32)

        pltpu.sync_copy(sl_t, deg_sh.at[pl.ds(nb, nodes_per)])
        plsc.subcore_barrier()

        @pl.loop(0, NSUP)
        def _(sc_i):
            pltpu.sync_copy(col_hbm.at[s, sc_i], col_s)
            pltpu.sync_copy(w_hbm.at[s, sc_i], w_s)
            for b in range(SB):
                pltpu.sync_copy(w_s.at[pl.ds(b * _CHUNK, _CHUNK)],
                                deg_sh.at[col_s.at[b]], add=True)

        plsc.subcore_barrier()

        pltpu.sync_copy(deg_sh.at[pl.ds(nb, nodes_per)], sl_t)

        @pl.loop(0, nodes_per, step=_L)
        def _(i):
            v = sl_t[pl.ds(i, _L)]
            bi = plsc.bitcast(v, jnp.int32)
            bi = jnp.full((_L,), 0x5F3759DF, jnp.int32) - lax.shift_right_logical(
                bi, jnp.full((_L,), 1, jnp.int32))
            y = plsc.bitcast(bi, jnp.float32)
            for _ in range(4):
                y = y * (1.5 - 0.5 * v * y * y)
            sl_t[pl.ds(i, _L)] = y

        pltpu.sync_copy(sl_t, dis_hbm.at[c, pl.ds(nb, nodes_per)])
        scale_slice(src_sh, squared=False)
        plsc.subcore_barrier()

        def hop(src, dst):
            @pl.loop(0, NSUP)
            def _(sc_i):
                pltpu.sync_copy(row_hbm.at[s, sc_i], row_s)
                pltpu.sync_copy(col_hbm.at[s, sc_i], col_s)
                pltpu.sync_copy(w_hbm.at[s, sc_i], w_s)
                sd = []
                for b in range(SB):
                    gslice = gbuf.at[pl.ds(b * _CHUNK, _CHUNK)]
                    pltpu.sync_copy(src.at[row_s.at[b]], gslice)

                    @pl.loop(0, _CHUNK)
                    def _(e2):
                        e = b * _CHUNK + e2
                        nv = plsc.load_gather(
                            w_s, [jnp.full((_L,), b * _CHUNK, jnp.int32) + e2])
                        for q in range(Dh // _L):
                            g = gbuf[e, pl.ds(q * _L, _L)]
                            gbuf[e, pl.ds(q * _L, _L)] = g * nv

                    sd.append(pltpu.async_copy(
                        gslice, dst.at[col_s.at[b]], ssems[b], add=True))
                for d in sd:
                    d.wait()

        hop(src_sh, acc_sh)
        plsc.subcore_barrier()
        scale_slice(acc_sh, squared=True)
        zero_gbuf_rows(_CHUNK)
        zero_spmem_slice(src_sh)
        plsc.subcore_barrier()
        hop(acc_sh, src_sh)
        plsc.subcore_barrier()
        pltpu.sync_copy(src_sh.at[pl.ds(nb, nodes_per)],
                        stage_hbm.at[pl.ds(sb_off, nodes_per)])

    cp = pltpu.CompilerParams()
    if "needs_layout_passes" in pltpu.CompilerParams.__dataclass_fields__:
        cp = dataclasses.replace(cp, needs_layout_passes=False)
    if "use_tc_tiling_on_sc" in pltpu.CompilerParams.__dataclass_fields__:
        cp = dataclasses.replace(cp, use_tc_tiling_on_sc=False)
    return pl.kernel(
        body,
        out_type=(jax.ShapeDtypeStruct((_NSC * N_pad, Dh), jnp.float32),
                  jax.ShapeDtypeStruct((_NSC, N_pad), jnp.float32)),
        mesh=mesh,
        compiler_params=cp,
        scratch_types=[
            pltpu.VMEM_SHARED((N_pad, Dh), jnp.float32),
            pltpu.VMEM_SHARED((N_pad, Dh), jnp.float32),
            pltpu.VMEM_SHARED((N_pad,), jnp.float32),
            pltpu.VMEM((SB, _CHUNK), jnp.int32),
            pltpu.VMEM((SB, _CHUNK), jnp.int32),
            pltpu.VMEM((SB * _CHUNK,), jnp.float32),
            pltpu.VMEM((N_pad // _NSUB,), jnp.float32),
            pltpu.VMEM((SB * _CHUNK, Dh), jnp.float32),
            [pltpu.SemaphoreType.DMA for _ in range(4)],
        ],
    )


def _tc_body(NB, hs_ref, b_ref, dis_ref, W1a_ref, W1b_ref, b1_ref, W2_ref,
             b2_ref, out_ref, acc_s, acc_c):
    i = pl.program_id(0)

    @pl.when(i == 0)
    def _():
        acc_s[...] = jnp.zeros_like(acc_s)
        acc_c[...] = jnp.zeros_like(acc_c)

    dv = dis_ref[0, 0, :][:, None]
    y = (jnp.dot(hs_ref[0] * dv, W1a_ref[...],
                 preferred_element_type=jnp.float32)
         + jnp.dot(hs_ref[1] * dv, W1b_ref[...],
                   preferred_element_type=jnp.float32))
    bt = b_ref[0, 0, :]
    onehot_t = (lax.broadcasted_iota(jnp.int32, (_G, bt.shape[0]), 0)
                == bt[None, :]).astype(jnp.float32)
    acc_s[...] += jnp.dot(onehot_t, y, preferred_element_type=jnp.float32)
    acc_c[...] += jnp.sum(onehot_t, axis=1, keepdims=True)

    @pl.when(i == NB - 1)
    def _():
        pooled = acc_s[...] / jnp.maximum(acc_c[...], 1.0) + b1_ref[...]
        out_ref[...] = (jnp.dot(pooled, W2_ref[...],
                                preferred_element_type=jnp.float32)
                        + b2_ref[...])


def kernel(x, edge_index, edge_attr, batch, W1, b1, W2, b2):
    N, D = x.shape
    E = edge_index.shape[1]
    H = W1.shape[1]
    P = W2.shape[1]
    Dh = D // 2

    N_pad = -(-N // 256) * 256
    SB = 4
    per = _NSUB * SB * _CHUNK
    E2 = E + N
    E2p = -(-E2 // per) * per
    NSUP = E2p // per

    loop = jnp.arange(N, dtype=jnp.int32)
    row = jnp.concatenate([edge_index[0], loop])
    col = jnp.concatenate([edge_index[1], loop])
    w = jnp.concatenate([edge_attr, jnp.ones((N,), x.dtype)])
    padn = E2p - E2
    shape4 = (_NSUB, NSUP, SB, _CHUNK)
    row4 = jnp.pad(row, (0, padn)).reshape(shape4)
    col4 = jnp.pad(col, (0, padn)).reshape(shape4)
    w3 = jnp.pad(w, (0, padn)).reshape(_NSUB, NSUP, SB * _CHUNK)

    x_pad = jnp.pad(x, ((0, N_pad - N), (0, 0)))
    xs = x_pad.reshape(N_pad, _NSC, Dh).transpose(1, 0, 2)

    stage, dis = _make_sc_propagate(N_pad, NSUP, SB, Dh)(xs, row4, col4, w3)
    h2 = stage.reshape(_NSC, N_pad, Dh)

    BN = 1024
    NB = N_pad // BN
    batch3 = jnp.pad(batch, (0, N_pad - N), constant_values=_G).reshape(
        NB, 1, BN)
    dis3 = dis[0].reshape(NB, 1, BN)
    W1a, W1b = W1[:Dh], W1[Dh:]

    out = pl.pallas_call(
        functools.partial(_tc_body, NB),
        grid=(NB,),
        in_specs=[
            pl.BlockSpec((_NSC, BN, Dh), lambda i: (0, i, 0)),
            pl.BlockSpec((1, 1, BN), lambda i: (i, 0, 0)),
            pl.BlockSpec((1, 1, BN), lambda i: (i, 0, 0)),
            pl.BlockSpec((Dh, H), lambda i: (0, 0)),
            pl.BlockSpec((Dh, H), lambda i: (0, 0)),
            pl.BlockSpec((1, H), lambda i: (0, 0)),
            pl.BlockSpec((H, P), lambda i: (0, 0)),
            pl.BlockSpec((1, P), lambda i: (0, 0)),
        ],
        out_specs=pl.BlockSpec((_G, P), lambda i: (0, 0)),
        out_shape=jax.ShapeDtypeStruct((_G, P), jnp.float32),
        scratch_shapes=[
            pltpu.VMEM((_G, H), jnp.float32),
            pltpu.VMEM((_G, 1), jnp.float32),
        ],
    )(h2, batch3, dis3, W1a, W1b, b1.reshape(1, H), W2, b2.reshape(1, P))
    return out


# speedup vs baseline: 2.3040x; 1.1036x over previous
import dataclasses
import functools

import jax
import jax.numpy as jnp
from jax import lax
from jax.experimental import pallas as pl
from jax.experimental.pallas import tpu as pltpu
from jax.experimental.pallas import tpu_sc as plsc

_NSC = 2
_NSUB = 16
_L = 16
_G = 128
_CHUNK = 128


def _make_sc_propagate(N_pad, NSUP, SB, Dh):
    nodes_per = N_pad // _NSUB
    mesh = plsc.VectorSubcoreMesh(
        core_axis_name="c", subcore_axis_name="s",
        num_cores=_NSC, num_subcores=_NSUB)

    def body(xs_hbm, row_hbm, col_hbm, w_hbm, stage_hbm, dis_hbm,
             src_sh, acc_sh, deg_sh,
             row_s, col_s, w_s, sl_t, gbuf, ssems, gsems):
        c = lax.axis_index("c")
        s = lax.axis_index("s")
        nb = s * nodes_per
        sb_off = c * N_pad + nb
        EPS = SB * _CHUNK

        def zero_gbuf_rows(n):
            @pl.loop(0, n)
            def _(i):
                for k in range(Dh // _L):
                    gbuf[i, pl.ds(k * _L, _L)] = jnp.zeros((_L,), jnp.float32)

        def zero_spmem_slice(dst):
            for k in range(nodes_per // _CHUNK):
                pltpu.sync_copy(gbuf.at[pl.ds(0, _CHUNK)],
                                dst.at[pl.ds(nb + k * _CHUNK, _CHUNK)])

        def scale_slice(dst, squared):
            for k in range(nodes_per // _CHUNK):
                pltpu.sync_copy(dst.at[pl.ds(nb + k * _CHUNK, _CHUNK)],
                                gbuf.at[pl.ds(0, _CHUNK)])

                @pl.loop(0, _CHUNK)
                def _(i):
                    dv = plsc.load_gather(
                        sl_t, [jnp.full((_L,), k * _CHUNK, jnp.int32) + i])
                    dv = dv * dv if squared else dv
                    for q in range(Dh // _L):
                        g = gbuf[i, pl.ds(q * _L, _L)]
                        gbuf[i, pl.ds(q * _L, _L)] = g * dv

                pltpu.sync_copy(gbuf.at[pl.ds(0, _CHUNK)],
                                dst.at[pl.ds(nb + k * _CHUNK, _CHUNK)])

        pltpu.sync_copy(xs_hbm.at[c, pl.ds(nb, nodes_per)],
                        src_sh.at[pl.ds(nb, nodes_per)])

        zero_gbuf_rows(_CHUNK)
        zero_spmem_slice(acc_sh)

        @pl.loop(0, nodes_per, step=_L)
        def _(i):
            sl_t[pl.ds(i, _L)] = jnp.zeros((_L,), jnp.float32)

        pltpu.sync_copy(sl_t, deg_sh.at[pl.ds(nb, nodes_per)])
        plsc.subcore_barrier()

        @pl.loop(0, NSUP)
        def _(sc_i):
            pltpu.sync_copy(col_hbm.at[s, sc_i], col_s)
            pltpu.sync_copy(w_hbm.at[s, sc_i], w_s)
            for b in range(SB):
                pltpu.sync_copy(w_s.at[pl.ds(b * _CHUNK, _CHUNK)],
                                deg_sh.at[col_s.at[b]], add=True)

        plsc.subcore_barrier()

        pltpu.sync_copy(deg_sh.at[pl.ds(nb, nodes_per)], sl_t)

        @pl.loop(0, nodes_per, step=_L)
        def _(i):
            v = sl_t[pl.ds(i, _L)]
            bi = plsc.bitcast(v, jnp.int32)
            bi = jnp.full((_L,), 0x5F3759DF, jnp.int32) - lax.shift_right_logical(
                bi, jnp.full((_L,), 1, jnp.int32))
            y = plsc.bitcast(bi, jnp.float32)
            for _ in range(4):
                y = y * (1.5 - 0.5 * v * y * y)
            sl_t[pl.ds(i, _L)] = y

        pltpu.sync_copy(sl_t, dis_hbm.at[c, pl.ds(nb, nodes_per)])
        scale_slice(src_sh, squared=False)
        plsc.subcore_barrier()

        def hop(src, dst):
            @pl.loop(0, NSUP)
            def _(sc_i):
                pltpu.sync_copy(row_hbm.at[s, sc_i], row_s)
                pltpu.sync_copy(col_hbm.at[s, sc_i], col_s)
                pltpu.sync_copy(w_hbm.at[s, sc_i], w_s)
                gd = []
                for b in range(SB):
                    gd.append(pltpu.async_copy(
                        src.at[row_s.at[b]],
                        gbuf.at[pl.ds(b * _CHUNK, _CHUNK)], gsems[b]))
                sd = []
                for b in range(SB):
                    gslice = gbuf.at[pl.ds(b * _CHUNK, _CHUNK)]
                    gd[b].wait()

                    @pl.loop(0, _CHUNK)
                    def _(e2):
                        e = b * _CHUNK + e2
                        nv = plsc.load_gather(
                            w_s, [jnp.full((_L,), b * _CHUNK, jnp.int32) + e2])
                        for q in range(Dh // _L):
                            g = gbuf[e, pl.ds(q * _L, _L)]
                            gbuf[e, pl.ds(q * _L, _L)] = g * nv

                    sd.append(pltpu.async_copy(
                        gslice, dst.at[col_s.at[b]], ssems[b], add=True))
                for d in sd:
                    d.wait()

        hop(src_sh, acc_sh)
        plsc.subcore_barrier()
        scale_slice(acc_sh, squared=True)
        zero_gbuf_rows(_CHUNK)
        zero_spmem_slice(src_sh)
        plsc.subcore_barrier()
        hop(acc_sh, src_sh)
        plsc.subcore_barrier()
        pltpu.sync_copy(src_sh.at[pl.ds(nb, nodes_per)],
                        stage_hbm.at[pl.ds(sb_off, nodes_per)])

    cp = pltpu.CompilerParams()
    if "needs_layout_passes" in pltpu.CompilerParams.__dataclass_fields__:
        cp = dataclasses.replace(cp, needs_layout_passes=False)
    if "use_tc_tiling_on_sc" in pltpu.CompilerParams.__dataclass_fields__:
        cp = dataclasses.replace(cp, use_tc_tiling_on_sc=False)
    return pl.kernel(
        body,
        out_type=(jax.ShapeDtypeStruct((_NSC * N_pad, Dh), jnp.float32),
                  jax.ShapeDtypeStruct((_NSC, N_pad), jnp.float32)),
        mesh=mesh,
        compiler_params=cp,
        scratch_types=[
            pltpu.VMEM_SHARED((N_pad, Dh), jnp.float32),
            pltpu.VMEM_SHARED((N_pad, Dh), jnp.float32),
            pltpu.VMEM_SHARED((N_pad,), jnp.float32),
            pltpu.VMEM((SB, _CHUNK), jnp.int32),
            pltpu.VMEM((SB, _CHUNK), jnp.int32),
            pltpu.VMEM((SB * _CHUNK,), jnp.float32),
            pltpu.VMEM((N_pad // _NSUB,), jnp.float32),
            pltpu.VMEM((SB * _CHUNK, Dh), jnp.float32),
            [pltpu.SemaphoreType.DMA for _ in range(4)],
            [pltpu.SemaphoreType.DMA for _ in range(4)],
        ],
    )


def _tc_body(NB, hs_ref, b_ref, dis_ref, W1a_ref, W1b_ref, b1_ref, W2_ref,
             b2_ref, out_ref, acc_s, acc_c):
    i = pl.program_id(0)

    @pl.when(i == 0)
    def _():
        acc_s[...] = jnp.zeros_like(acc_s)
        acc_c[...] = jnp.zeros_like(acc_c)

    dv = dis_ref[0, 0, :][:, None]
    y = (jnp.dot(hs_ref[0] * dv, W1a_ref[...],
                 preferred_element_type=jnp.float32)
         + jnp.dot(hs_ref[1] * dv, W1b_ref[...],
                   preferred_element_type=jnp.float32))
    bt = b_ref[0, 0, :]
    onehot_t = (lax.broadcasted_iota(jnp.int32, (_G, bt.shape[0]), 0)
                == bt[None, :]).astype(jnp.float32)
    acc_s[...] += jnp.dot(onehot_t, y, preferred_element_type=jnp.float32)
    acc_c[...] += jnp.sum(onehot_t, axis=1, keepdims=True)

    @pl.when(i == NB - 1)
    def _():
        pooled = acc_s[...] / jnp.maximum(acc_c[...], 1.0) + b1_ref[...]
        out_ref[...] = (jnp.dot(pooled, W2_ref[...],
                                preferred_element_type=jnp.float32)
                        + b2_ref[...])


def kernel(x, edge_index, edge_attr, batch, W1, b1, W2, b2):
    N, D = x.shape
    E = edge_index.shape[1]
    H = W1.shape[1]
    P = W2.shape[1]
    Dh = D // 2

    N_pad = -(-N // 256) * 256
    SB = 4
    per = _NSUB * SB * _CHUNK
    E2 = E + N
    E2p = -(-E2 // per) * per
    NSUP = E2p // per

    loop = jnp.arange(N, dtype=jnp.int32)
    row = jnp.concatenate([edge_index[0], loop])
    col = jnp.concatenate([edge_index[1], loop])
    w = jnp.concatenate([edge_attr, jnp.ones((N,), x.dtype)])
    padn = E2p - E2
    shape4 = (_NSUB, NSUP, SB, _CHUNK)
    row4 = jnp.pad(row, (0, padn)).reshape(shape4)
    col4 = jnp.pad(col, (0, padn)).reshape(shape4)
    w3 = jnp.pad(w, (0, padn)).reshape(_NSUB, NSUP, SB * _CHUNK)

    x_pad = jnp.pad(x, ((0, N_pad - N), (0, 0)))
    xs = x_pad.reshape(N_pad, _NSC, Dh).transpose(1, 0, 2)

    stage, dis = _make_sc_propagate(N_pad, NSUP, SB, Dh)(xs, row4, col4, w3)
    h2 = stage.reshape(_NSC, N_pad, Dh)

    BN = 1024
    NB = N_pad // BN
    batch3 = jnp.pad(batch, (0, N_pad - N), constant_values=_G).reshape(
        NB, 1, BN)
    dis3 = dis[0].reshape(NB, 1, BN)
    W1a, W1b = W1[:Dh], W1[Dh:]

    out = pl.pallas_call(
        functools.partial(_tc_body, NB),
        grid=(NB,),
        in_specs=[
            pl.BlockSpec((_NSC, BN, Dh), lambda i: (0, i, 0)),
            pl.BlockSpec((1, 1, BN), lambda i: (i, 0, 0)),
            pl.BlockSpec((1, 1, BN), lambda i: (i, 0, 0)),
            pl.BlockSpec((Dh, H), lambda i: (0, 0)),
            pl.BlockSpec((Dh, H), lambda i: (0, 0)),
            pl.BlockSpec((1, H), lambda i: (0, 0)),
            pl.BlockSpec((H, P), lambda i: (0, 0)),
            pl.BlockSpec((1, P), lambda i: (0, 0)),
        ],
        out_specs=pl.BlockSpec((_G, P), lambda i: (0, 0)),
        out_shape=jax.ShapeDtypeStruct((_G, P), jnp.float32),
        scratch_shapes=[
            pltpu.VMEM((_G, H), jnp.float32),
            pltpu.VMEM((_G, 1), jnp.float32),
        ],
    )(h2, batch3, dis3, W1a, W1b, b1.reshape(1, H), W2, b2.reshape(1, P))
    return out


# speedup vs baseline: 2.3216x; 1.0077x over previous
import dataclasses
import functools

import jax
import jax.numpy as jnp
from jax import lax
from jax.experimental import pallas as pl
from jax.experimental.pallas import tpu as pltpu
from jax.experimental.pallas import tpu_sc as plsc

_NSC = 2
_NSUB = 16
_L = 16
_G = 128
_CHUNK = 128


def _make_sc_propagate(N_pad, NSUP, SB, Dh):
    nodes_per = N_pad // _NSUB
    mesh = plsc.VectorSubcoreMesh(
        core_axis_name="c", subcore_axis_name="s",
        num_cores=_NSC, num_subcores=_NSUB)

    def body(xs_hbm, row_hbm, col_hbm, w_hbm, stage_hbm, dis_hbm,
             src_sh, acc_sh, deg_sh,
             row_s, col_s, w_s, sl_t, gbuf, ssems, gsems):
        c = lax.axis_index("c")
        s = lax.axis_index("s")
        nb = s * nodes_per
        sb_off = c * N_pad + nb
        EPS = SB * _CHUNK

        def zero_gbuf_rows(n):
            @pl.loop(0, n)
            def _(i):
                for k in range(Dh // _L):
                    gbuf[i, pl.ds(k * _L, _L)] = jnp.zeros((_L,), jnp.float32)

        def zero_spmem_slice(dst):
            for k in range(nodes_per // _CHUNK):
                pltpu.sync_copy(gbuf.at[pl.ds(0, _CHUNK)],
                                dst.at[pl.ds(nb + k * _CHUNK, _CHUNK)])

        def scale_slice(dst, squared):
            for k in range(nodes_per // _CHUNK):
                pltpu.sync_copy(dst.at[pl.ds(nb + k * _CHUNK, _CHUNK)],
                                gbuf.at[pl.ds(0, _CHUNK)])

                @pl.loop(0, _CHUNK)
                def _(i):
                    dv = plsc.load_gather(
                        sl_t, [jnp.full((_L,), k * _CHUNK, jnp.int32) + i])
                    dv = dv * dv if squared else dv
                    for q in range(Dh // _L):
                        g = gbuf[i, pl.ds(q * _L, _L)]
                        gbuf[i, pl.ds(q * _L, _L)] = g * dv

                pltpu.sync_copy(gbuf.at[pl.ds(0, _CHUNK)],
                                dst.at[pl.ds(nb + k * _CHUNK, _CHUNK)])

        pltpu.sync_copy(xs_hbm.at[c, pl.ds(nb, nodes_per)],
                        src_sh.at[pl.ds(nb, nodes_per)])

        zero_gbuf_rows(_CHUNK)
        zero_spmem_slice(acc_sh)

        @pl.loop(0, nodes_per, step=_L)
        def _(i):
            sl_t[pl.ds(i, _L)] = jnp.zeros((_L,), jnp.float32)

        pltpu.sync_copy(sl_t, deg_sh.at[pl.ds(nb, nodes_per)])
        plsc.subcore_barrier()

        @pl.loop(0, NSUP)
        def _(sc_i):
            pltpu.sync_copy(col_hbm.at[s, sc_i], col_s)
            pltpu.sync_copy(w_hbm.at[s, sc_i], w_s)
            for b in range(SB):
                pltpu.sync_copy(w_s.at[pl.ds(b * _CHUNK, _CHUNK)],
                                deg_sh.at[col_s.at[b]], add=True)

        plsc.subcore_barrier()

        pltpu.sync_copy(deg_sh.at[pl.ds(nb, nodes_per)], sl_t)

        @pl.loop(0, nodes_per, step=_L)
        def _(i):
            v = sl_t[pl.ds(i, _L)]
            bi = plsc.bitcast(v, jnp.int32)
            bi = jnp.full((_L,), 0x5F3759DF, jnp.int32) - lax.shift_right_logical(
                bi, jnp.full((_L,), 1, jnp.int32))
            y = plsc.bitcast(bi, jnp.float32)
            for _ in range(4):
                y = y * (1.5 - 0.5 * v * y * y)
            sl_t[pl.ds(i, _L)] = y

        pltpu.sync_copy(sl_t, dis_hbm.at[c, pl.ds(nb, nodes_per)])
        scale_slice(src_sh, squared=False)
        plsc.subcore_barrier()

        def hop(src, dst):
            @pl.loop(0, NSUP)
            def _(sc_i):
                pltpu.sync_copy(row_hbm.at[s, sc_i], row_s)
                pltpu.sync_copy(col_hbm.at[s, sc_i], col_s)
                pltpu.sync_copy(w_hbm.at[s, sc_i], w_s)
                gd = []
                for b in range(SB):
                    gd.append(pltpu.async_copy(
                        src.at[row_s.at[b]],
                        gbuf.at[pl.ds(b * _CHUNK, _CHUNK)], gsems[b]))
                sd = []
                for b in range(SB):
                    gslice = gbuf.at[pl.ds(b * _CHUNK, _CHUNK)]
                    gd[b].wait()

                    @pl.loop(0, _CHUNK, step=4)
                    def _(e2):
                        for j in range(4):
                            e = b * _CHUNK + e2 + j
                            nv = plsc.load_gather(
                                w_s, [jnp.full((_L,), b * _CHUNK + j,
                                               jnp.int32) + e2])
                            for q in range(Dh // _L):
                                g = gbuf[e, pl.ds(q * _L, _L)]
                                gbuf[e, pl.ds(q * _L, _L)] = g * nv

                    sd.append(pltpu.async_copy(
                        gslice, dst.at[col_s.at[b]], ssems[b], add=True))
                for d in sd:
                    d.wait()

        hop(src_sh, acc_sh)
        plsc.subcore_barrier()
        scale_slice(acc_sh, squared=True)
        zero_gbuf_rows(_CHUNK)
        zero_spmem_slice(src_sh)
        plsc.subcore_barrier()
        hop(acc_sh, src_sh)
        plsc.subcore_barrier()
        pltpu.sync_copy(src_sh.at[pl.ds(nb, nodes_per)],
                        stage_hbm.at[pl.ds(sb_off, nodes_per)])

    cp = pltpu.CompilerParams()
    if "needs_layout_passes" in pltpu.CompilerParams.__dataclass_fields__:
        cp = dataclasses.replace(cp, needs_layout_passes=False)
    if "use_tc_tiling_on_sc" in pltpu.CompilerParams.__dataclass_fields__:
        cp = dataclasses.replace(cp, use_tc_tiling_on_sc=False)
    return pl.kernel(
        body,
        out_type=(jax.ShapeDtypeStruct((_NSC * N_pad, Dh), jnp.float32),
                  jax.ShapeDtypeStruct((_NSC, N_pad), jnp.float32)),
        mesh=mesh,
        compiler_params=cp,
        scratch_types=[
            pltpu.VMEM_SHARED((N_pad, Dh), jnp.float32),
            pltpu.VMEM_SHARED((N_pad, Dh), jnp.float32),
            pltpu.VMEM_SHARED((N_pad,), jnp.float32),
            pltpu.VMEM((SB, _CHUNK), jnp.int32),
            pltpu.VMEM((SB, _CHUNK), jnp.int32),
            pltpu.VMEM((SB * _CHUNK,), jnp.float32),
            pltpu.VMEM((N_pad // _NSUB,), jnp.float32),
            pltpu.VMEM((SB * _CHUNK, Dh), jnp.float32),
            [pltpu.SemaphoreType.DMA for _ in range(4)],
            [pltpu.SemaphoreType.DMA for _ in range(4)],
        ],
    )


def _tc_body(NB, hs_ref, b_ref, dis_ref, W1a_ref, W1b_ref, b1_ref, W2_ref,
             b2_ref, out_ref, acc_s, acc_c):
    i = pl.program_id(0)

    @pl.when(i == 0)
    def _():
        acc_s[...] = jnp.zeros_like(acc_s)
        acc_c[...] = jnp.zeros_like(acc_c)

    dv = dis_ref[0, 0, :][:, None]
    y = (jnp.dot(hs_ref[0] * dv, W1a_ref[...],
                 preferred_element_type=jnp.float32)
         + jnp.dot(hs_ref[1] * dv, W1b_ref[...],
                   preferred_element_type=jnp.float32))
    bt = b_ref[0, 0, :]
    onehot_t = (lax.broadcasted_iota(jnp.int32, (_G, bt.shape[0]), 0)
                == bt[None, :]).astype(jnp.float32)
    acc_s[...] += jnp.dot(onehot_t, y, preferred_element_type=jnp.float32)
    acc_c[...] += jnp.sum(onehot_t, axis=1, keepdims=True)

    @pl.when(i == NB - 1)
    def _():
        pooled = acc_s[...] / jnp.maximum(acc_c[...], 1.0) + b1_ref[...]
        out_ref[...] = (jnp.dot(pooled, W2_ref[...],
                                preferred_element_type=jnp.float32)
                        + b2_ref[...])


def kernel(x, edge_index, edge_attr, batch, W1, b1, W2, b2):
    N, D = x.shape
    E = edge_index.shape[1]
    H = W1.shape[1]
    P = W2.shape[1]
    Dh = D // 2

    N_pad = -(-N // 256) * 256
    SB = 4
    per = _NSUB * SB * _CHUNK
    E2 = E + N
    E2p = -(-E2 // per) * per
    NSUP = E2p // per

    loop = jnp.arange(N, dtype=jnp.int32)
    row = jnp.concatenate([edge_index[0], loop])
    col = jnp.concatenate([edge_index[1], loop])
    w = jnp.concatenate([edge_attr, jnp.ones((N,), x.dtype)])
    padn = E2p - E2
    shape4 = (_NSUB, NSUP, SB, _CHUNK)
    row4 = jnp.pad(row, (0, padn)).reshape(shape4)
    col4 = jnp.pad(col, (0, padn)).reshape(shape4)
    w3 = jnp.pad(w, (0, padn)).reshape(_NSUB, NSUP, SB * _CHUNK)

    x_pad = jnp.pad(x, ((0, N_pad - N), (0, 0)))
    xs = x_pad.reshape(N_pad, _NSC, Dh).transpose(1, 0, 2)

    stage, dis = _make_sc_propagate(N_pad, NSUP, SB, Dh)(xs, row4, col4, w3)
    h2 = stage.reshape(_NSC, N_pad, Dh)

    BN = 1024
    NB = N_pad // BN
    batch3 = jnp.pad(batch, (0, N_pad - N), constant_values=_G).reshape(
        NB, 1, BN)
    dis3 = dis[0].reshape(NB, 1, BN)
    W1a, W1b = W1[:Dh], W1[Dh:]

    out = pl.pallas_call(
        functools.partial(_tc_body, NB),
        grid=(NB,),
        in_specs=[
            pl.BlockSpec((_NSC, BN, Dh), lambda i: (0, i, 0)),
            pl.BlockSpec((1, 1, BN), lambda i: (i, 0, 0)),
            pl.BlockSpec((1, 1, BN), lambda i: (i, 0, 0)),
            pl.BlockSpec((Dh, H), lambda i: (0, 0)),
            pl.BlockSpec((Dh, H), lambda i: (0, 0)),
            pl.BlockSpec((1, H), lambda i: (0, 0)),
            pl.BlockSpec((H, P), lambda i: (0, 0)),
            pl.BlockSpec((1, P), lambda i: (0, 0)),
        ],
        out_specs=pl.BlockSpec((_G, P), lambda i: (0, 0)),
        out_shape=jax.ShapeDtypeStruct((_G, P), jnp.float32),
        scratch_shapes=[
            pltpu.VMEM((_G, H), jnp.float32),
            pltpu.VMEM((_G, 1), jnp.float32),
        ],
    )(h2, batch3, dis3, W1a, W1b, b1.reshape(1, H), W2, b2.reshape(1, P))
    return out
